# Initial kernel scaffold; baseline (speedup 1.0000x reference)
import jax
import jax.numpy as jnp
from jax.experimental import pallas as pl


def kernel(node_features, node_attrs, edge_embedding, edge_attrs, edge_index, W1, fc_w0, fc_w1, W2, W_sc):
    raise NotImplementedError("write your pallas kernel here")


# speedup vs baseline: 2.4266x; 2.4266x over previous
import functools
import math

import jax
import jax.numpy as jnp
from jax import lax
from jax.experimental import pallas as pl
from jax.experimental.pallas import tpu as pltpu
from jax.experimental.pallas import tpu_sc as plsc

N = 10000
E = 320000
F = 128
A = 16
D_EDGE = 16
H = 8
AVG_NEIGHBORS = 32.0

NC = 2
NS = 16
NW = NC * NS
CHUNK = 64
CPW = 160
E_PW = CPW * CHUNK
E_PAD = NW * E_PW
N_PAD = 10240
ROWS_PT = N_PAD // NS

_INV_SQRT = 1.0 / math.sqrt(AVG_NEIGHBORS)


def _h_body(nf_ref, w_ref, o_ref):
    o_ref[...] = jnp.dot(nf_ref[...], w_ref[...], preferred_element_type=jnp.float32)


def _h_matmul(nf, W1):
    BN = 2000
    return pl.pallas_call(
        _h_body,
        grid=(N // BN,),
        in_specs=[
            pl.BlockSpec((BN, F), lambda i: (i, 0)),
            pl.BlockSpec((F, F), lambda i: (0, 0)),
        ],
        out_specs=pl.BlockSpec((BN, F), lambda i: (i, 0)),
        out_shape=jax.ShapeDtypeStruct((N, F), jnp.float32),
    )(nf, W1)


def _m_body(ee_ref, a_ref, w0_ref, w1_ref, o_ref):
    u = jnp.dot(ee_ref[...], w0_ref[...], preferred_element_type=jnp.float32)
    u = u * jax.nn.sigmoid(u)
    w = jnp.dot(u, w1_ref[...], preferred_element_type=jnp.float32)
    o_ref[...] = w * a_ref[...]


def _edge_coeffs(ee_pad, attrs_pad, fc_w0, fc_w1):
    BE = 2048
    return pl.pallas_call(
        _m_body,
        grid=(E_PAD // BE,),
        in_specs=[
            pl.BlockSpec((BE, D_EDGE), lambda i: (i, 0)),
            pl.BlockSpec((BE, 1), lambda i: (i, 0)),
            pl.BlockSpec((D_EDGE, H), lambda i: (0, 0)),
            pl.BlockSpec((H, F), lambda i: (0, 0)),
        ],
        out_specs=pl.BlockSpec((BE, F), lambda i: (i, 0)),
        out_shape=jax.ShapeDtypeStruct((E_PAD, F), jnp.float32),
    )(ee_pad, attrs_pad, fc_w0, fc_w1)


def _sc_body(h_hbm, m_hbm, src_hbm, dst_hbm, out_hbm,
             src_v, dst_v, rows_v, mv, acc_sh,
             gsem0, gsem1, msem0, msem1, isem0, isem1, isem2, isem3):
    c = lax.axis_index("c")
    s = lax.axis_index("s")
    wid = s * NC + c
    base_e = wid * E_PW
    base_n = s * ROWS_PT
    gsems = (gsem0, gsem1)
    msems = (msem0, msem1)
    isems = (isem0, isem1, isem2, isem3)
    zero16 = jnp.zeros((16,), jnp.float32)

    def _zrow(r, _):
        for f16 in range(F // 16):
            rows_v[0, r, pl.ds(f16 * 16, 16)] = zero16
        return 0

    lax.fori_loop(0, CHUNK, _zrow, 0)
    for k in range(ROWS_PT // CHUNK):
        pltpu.sync_copy(rows_v.at[0],
                        acc_sh.at[pl.ds(base_n + k * CHUNK, CHUNK)])
    plsc.subcore_barrier()

    def _idx_start(j, slot):
        pltpu.async_copy(src_hbm.at[pl.ds(base_e + j * CHUNK, CHUNK)],
                         src_v.at[slot], isems[slot])
        pltpu.async_copy(dst_hbm.at[pl.ds(base_e + j * CHUNK, CHUNK)],
                         dst_v.at[slot], isems[slot])

    def _idx_wait(j, slot):
        pltpu.make_async_copy(src_hbm.at[pl.ds(base_e + j * CHUNK, CHUNK)],
                              src_v.at[slot], isems[slot]).wait()
        pltpu.make_async_copy(dst_hbm.at[pl.ds(base_e + j * CHUNK, CHUNK)],
                              dst_v.at[slot], isems[slot]).wait()

    def _start(j, b, slot):
        pltpu.async_copy(h_hbm.at[src_v.at[slot]], rows_v.at[b], gsems[b])
        pltpu.async_copy(m_hbm.at[pl.ds(base_e + j * CHUNK, CHUNK)],
                         mv.at[b], msems[b])

    def _wait(j, b):
        pltpu.make_async_copy(h_hbm.at[src_v.at[0]], rows_v.at[b],
                              gsems[b]).wait()
        pltpu.make_async_copy(m_hbm.at[pl.ds(base_e + j * CHUNK, CHUNK)],
                              mv.at[b], msems[b]).wait()

    _idx_start(0, 0)
    _idx_start(1, 1)
    _idx_wait(0, 0)
    _start(0, 0, 0)
    _idx_start(2, 2)
    _idx_start(3, 3)

    def _chunk_quad(jq, _):
        for q in (0, 1, 2, 3):
            j = jq * 4 + q
            b = q % 2
            nb = 1 - b
            slot = q
            nslot = (q + 1) % 4

            @pl.when(j + 1 < CPW)
            def _():
                _idx_wait(j + 1, nslot)
                _start(j + 1, nb, nslot)

            _wait(j, b)

            def _mul(r, _c):
                for f16 in range(F // 16):
                    sl = pl.ds(f16 * 16, 16)
                    rows_v[b, r, sl] = rows_v[b, r, sl] * mv[b, r, sl]
                return 0

            lax.fori_loop(0, CHUNK, _mul, 0)
            pltpu.sync_copy(rows_v.at[b], acc_sh.at[dst_v.at[slot]], add=True)
            @pl.when(j + 4 < CPW)
            def _():
                _idx_start(j + 4, slot)
        return 0

    lax.fori_loop(0, CPW // 4, _chunk_quad, 0)

    plsc.subcore_barrier()
    pltpu.sync_copy(acc_sh.at[pl.ds(base_n, ROWS_PT)],
                    out_hbm.at[pl.ds(c * N_PAD + base_n, ROWS_PT)])


def _sc_conv(h, m, src1, dst1):
    mesh = plsc.VectorSubcoreMesh(core_axis_name="c", subcore_axis_name="s")
    f = functools.partial(
        pl.kernel,
        out_type=jax.ShapeDtypeStruct((NC * N_PAD, F), jnp.float32),
        mesh=mesh,
        scratch_types=[
            pltpu.VMEM((4, CHUNK), jnp.int32),
            pltpu.VMEM((4, CHUNK), jnp.int32),
            pltpu.VMEM((2, CHUNK, F), jnp.float32),
            pltpu.VMEM((2, CHUNK, F), jnp.float32),
            pltpu.VMEM_SHARED((N_PAD, F), jnp.float32),
            pltpu.SemaphoreType.DMA,
            pltpu.SemaphoreType.DMA,
            pltpu.SemaphoreType.DMA,
            pltpu.SemaphoreType.DMA,
            pltpu.SemaphoreType.DMA,
            pltpu.SemaphoreType.DMA,
            pltpu.SemaphoreType.DMA,
            pltpu.SemaphoreType.DMA,
        ],
    )(_sc_body)
    return f(h, m, src1, dst1)


def _final_body(p0_ref, p1_ref, nf_ref, na_ref, w2_ref, wsc_ref, o_ref):
    msg = (p0_ref[0] + p1_ref[0]) * _INV_SQRT
    acc = jnp.dot(msg, w2_ref[...], preferred_element_type=jnp.float32)
    nfb = nf_ref[...]
    nab = na_ref[...]
    for a in range(A):
        acc = acc + jnp.dot(nfb * nab[:, a:a + 1], wsc_ref[a],
                            preferred_element_type=jnp.float32)
    o_ref[...] = acc


def _final(partial2, nf, na, W2, Wsc_t):
    BN = 2000
    return pl.pallas_call(
        _final_body,
        grid=(N // BN,),
        in_specs=[
            pl.BlockSpec((1, BN, F), lambda i: (0, i, 0)),
            pl.BlockSpec((1, BN, F), lambda i: (1, i, 0)),
            pl.BlockSpec((BN, F), lambda i: (i, 0)),
            pl.BlockSpec((BN, A), lambda i: (i, 0)),
            pl.BlockSpec((F, F), lambda i: (0, 0)),
            pl.BlockSpec((A, F, F), lambda i: (0, 0, 0)),
        ],
        out_specs=pl.BlockSpec((BN, F), lambda i: (i, 0)),
        out_shape=jax.ShapeDtypeStruct((N, F), jnp.float32),
    )(partial2, partial2, nf, na, W2, Wsc_t)


def kernel(node_features, node_attrs, edge_embedding, edge_attrs, edge_index,
           W1, fc_w0, fc_w1, W2, W_sc):
    pad = E_PAD - E
    ee_pad = jnp.pad(edge_embedding, ((0, pad), (0, 0)))
    attrs_pad = jnp.pad(edge_attrs, ((0, pad), (0, 0)))
    filler = (jnp.arange(pad, dtype=jnp.int32) * 131) % N
    src1 = jnp.concatenate([edge_index[0], filler])
    dst1 = jnp.concatenate([edge_index[1], filler])

    h = _h_matmul(node_features, W1)
    m = _edge_coeffs(ee_pad, attrs_pad, fc_w0, fc_w1)
    partial2 = _sc_conv(h, m, src1, dst1).reshape(NC, N_PAD, F)
    Wsc_t = W_sc.transpose(1, 0, 2)
    return _final(partial2, node_features, node_attrs, W2, Wsc_t)


# speedup vs baseline: 2.9534x; 1.2171x over previous
import functools
import math

import jax
import jax.numpy as jnp
from jax import lax
from jax.experimental import pallas as pl
from jax.experimental.pallas import tpu as pltpu
from jax.experimental.pallas import tpu_sc as plsc

N = 10000
E = 320000
F = 128
A = 16
D_EDGE = 16
H = 8
AVG_NEIGHBORS = 32.0

NC = 2
NS = 16
NW = NC * NS
CHUNK = 64
CPW = 160
E_PW = CPW * CHUNK
CPW_LAST = (E - (NW - 1) * E_PW) // CHUNK
N_PAD = 10240
ROWS_PT = N_PAD // NS

_INV_SQRT = 1.0 / math.sqrt(AVG_NEIGHBORS)


def _h_body(nf_ref, w_ref, o_ref):
    o_ref[...] = jnp.dot(nf_ref[...], w_ref[...], preferred_element_type=jnp.float32)


def _h_matmul(nf, W1):
    BN = 2000
    return pl.pallas_call(
        _h_body,
        grid=(N // BN,),
        in_specs=[
            pl.BlockSpec((BN, F), lambda i: (i, 0)),
            pl.BlockSpec((F, F), lambda i: (0, 0)),
        ],
        out_specs=pl.BlockSpec((BN, F), lambda i: (i, 0)),
        out_shape=jax.ShapeDtypeStruct((N, F), jnp.float32),
    )(nf, W1)


def _m_body(ee_ref, a_ref, w0_ref, w1_ref, o_ref):
    u = jnp.dot(ee_ref[...], w0_ref[...], preferred_element_type=jnp.float32)
    u = u * jax.nn.sigmoid(u)
    w = jnp.dot(u, w1_ref[...], preferred_element_type=jnp.float32)
    o_ref[...] = w * a_ref[...]


def _edge_coeffs(ee, attrs, fc_w0, fc_w1):
    BE = 2000
    return pl.pallas_call(
        _m_body,
        grid=(E // BE,),
        in_specs=[
            pl.BlockSpec((BE, D_EDGE), lambda i: (i, 0)),
            pl.BlockSpec((BE, 1), lambda i: (i, 0)),
            pl.BlockSpec((D_EDGE, H), lambda i: (0, 0)),
            pl.BlockSpec((H, F), lambda i: (0, 0)),
        ],
        out_specs=pl.BlockSpec((BE, F), lambda i: (i, 0)),
        out_shape=jax.ShapeDtypeStruct((E, F), jnp.float32),
    )(ee, attrs, fc_w0, fc_w1)


def _sc_body(h_hbm, m_hbm, src_hbm, dst_hbm, out_hbm,
             src_v, dst_v, rows_v, mv, acc_sh,
             gsem0, gsem1, msem0, msem1, isem0, isem1, isem2, isem3):
    c = lax.axis_index("c")
    s = lax.axis_index("s")
    wid = s * NC + c
    base_e = wid * E_PW
    base_n = s * ROWS_PT
    gsems = (gsem0, gsem1)
    msems = (msem0, msem1)
    isems = (isem0, isem1, isem2, isem3)
    zero16 = jnp.zeros((16,), jnp.float32)
    rc = jnp.minimum(lax.div(E - wid * E_PW, CHUNK), CPW)

    def _zrow(r, _):
        for f16 in range(F // 16):
            rows_v[0, r, pl.ds(f16 * 16, 16)] = zero16
        return 0

    lax.fori_loop(0, CHUNK, _zrow, 0)
    for k in range(ROWS_PT // CHUNK):
        pltpu.sync_copy(rows_v.at[0],
                        acc_sh.at[pl.ds(base_n + k * CHUNK, CHUNK)])
    plsc.subcore_barrier()

    def _idx_start(j, slot):
        pltpu.async_copy(src_hbm.at[pl.ds(base_e + j * CHUNK, CHUNK)],
                         src_v.at[slot], isems[slot])
        pltpu.async_copy(dst_hbm.at[pl.ds(base_e + j * CHUNK, CHUNK)],
                         dst_v.at[slot], isems[slot])

    def _idx_wait(j, slot):
        pltpu.make_async_copy(src_hbm.at[pl.ds(base_e + j * CHUNK, CHUNK)],
                              src_v.at[slot], isems[slot]).wait()
        pltpu.make_async_copy(dst_hbm.at[pl.ds(base_e + j * CHUNK, CHUNK)],
                              dst_v.at[slot], isems[slot]).wait()

    def _start(j, b, slot):
        pltpu.async_copy(h_hbm.at[src_v.at[slot]], rows_v.at[b], gsems[b])
        pltpu.async_copy(m_hbm.at[pl.ds(base_e + j * CHUNK, CHUNK)],
                         mv.at[b], msems[b])

    def _wait(j, b):
        pltpu.make_async_copy(h_hbm.at[src_v.at[0]], rows_v.at[b],
                              gsems[b]).wait()
        pltpu.make_async_copy(m_hbm.at[pl.ds(base_e + j * CHUNK, CHUNK)],
                              mv.at[b], msems[b]).wait()

    _idx_start(0, 0)
    _idx_start(1, 1)
    _idx_wait(0, 0)
    _start(0, 0, 0)
    _idx_start(2, 2)
    _idx_start(3, 3)

    def _chunk_quad(jq, _):
        for q in (0, 1, 2, 3):
            j = jq * 4 + q
            b = q % 2
            nb = 1 - b
            slot = q
            nslot = (q + 1) % 4

            @pl.when(j + 1 < rc)
            def _():
                _idx_wait(j + 1, nslot)
                _start(j + 1, nb, nslot)

            _wait(j, b)

            def _mul(r, _c):
                for f16 in range(F // 16):
                    sl = pl.ds(f16 * 16, 16)
                    rows_v[b, r, sl] = rows_v[b, r, sl] * mv[b, r, sl]
                return 0

            lax.fori_loop(0, CHUNK, _mul, 0)
            pltpu.sync_copy(rows_v.at[b], acc_sh.at[dst_v.at[slot]], add=True)
            @pl.when(j + 4 < rc)
            def _():
                _idx_start(j + 4, slot)
        return 0

    lax.fori_loop(0, rc // 4, _chunk_quad, 0)

    plsc.subcore_barrier()
    pltpu.sync_copy(acc_sh.at[pl.ds(base_n, ROWS_PT)],
                    out_hbm.at[pl.ds(c * N_PAD + base_n, ROWS_PT)])


def _sc_conv(h, m, src1, dst1):
    mesh = plsc.VectorSubcoreMesh(core_axis_name="c", subcore_axis_name="s")
    f = functools.partial(
        pl.kernel,
        out_type=jax.ShapeDtypeStruct((NC * N_PAD, F), jnp.float32),
        mesh=mesh,
        scratch_types=[
            pltpu.VMEM((4, CHUNK), jnp.int32),
            pltpu.VMEM((4, CHUNK), jnp.int32),
            pltpu.VMEM((2, CHUNK, F), jnp.float32),
            pltpu.VMEM((2, CHUNK, F), jnp.float32),
            pltpu.VMEM_SHARED((N_PAD, F), jnp.float32),
            pltpu.SemaphoreType.DMA,
            pltpu.SemaphoreType.DMA,
            pltpu.SemaphoreType.DMA,
            pltpu.SemaphoreType.DMA,
            pltpu.SemaphoreType.DMA,
            pltpu.SemaphoreType.DMA,
            pltpu.SemaphoreType.DMA,
            pltpu.SemaphoreType.DMA,
        ],
    )(_sc_body)
    return f(h, m, src1, dst1)


def _final_body(p0_ref, p1_ref, nf_ref, na_ref, w2_ref, wsc_ref, o_ref):
    msg = (p0_ref[0] + p1_ref[0]) * _INV_SQRT
    acc = jnp.dot(msg, w2_ref[...], preferred_element_type=jnp.float32)
    nfb = nf_ref[...]
    nab = na_ref[...]
    for a in range(A):
        acc = acc + jnp.dot(nfb * nab[:, a:a + 1], wsc_ref[a],
                            preferred_element_type=jnp.float32)
    o_ref[...] = acc


def _final(partial2, nf, na, W2, Wsc_t):
    BN = 2000
    return pl.pallas_call(
        _final_body,
        grid=(N // BN,),
        in_specs=[
            pl.BlockSpec((1, BN, F), lambda i: (0, i, 0)),
            pl.BlockSpec((1, BN, F), lambda i: (1, i, 0)),
            pl.BlockSpec((BN, F), lambda i: (i, 0)),
            pl.BlockSpec((BN, A), lambda i: (i, 0)),
            pl.BlockSpec((F, F), lambda i: (0, 0)),
            pl.BlockSpec((A, F, F), lambda i: (0, 0, 0)),
        ],
        out_specs=pl.BlockSpec((BN, F), lambda i: (i, 0)),
        out_shape=jax.ShapeDtypeStruct((N, F), jnp.float32),
    )(partial2, partial2, nf, na, W2, Wsc_t)


_PERM = []
for _k in range(F // 32):
    _PERM.extend(16 * _k + _i for _i in range(16))
    _PERM.extend(64 + 16 * _k + _i for _i in range(16))


def kernel(node_features, node_attrs, edge_embedding, edge_attrs, edge_index,
           W1, fc_w0, fc_w1, W2, W_sc):
    h = _h_matmul(node_features, W1)
    m = _edge_coeffs(edge_embedding, edge_attrs, fc_w0, fc_w1)
    partial2 = _sc_conv(h, m, edge_index[0], edge_index[1]).reshape(NC, N_PAD, F)
    Wsc_t = W_sc.transpose(1, 0, 2)
    return _final(partial2, node_features, node_attrs, W2, Wsc_t)
